# Initial kernel scaffold; baseline (speedup 1.0000x reference)
import jax
import jax.numpy as jnp
from jax.experimental import pallas as pl


def kernel(x, conv0_w, conv0_scale, conv0_shift, conv1_w, conv1_scale, conv1_shift, conv2_w, conv2_scale, conv2_shift, conv3_w, conv3_scale, conv3_shift, conv4_w, conv4_scale, conv4_shift, conv5_w, conv5_scale, conv5_shift, conv6_w, conv6_scale, conv6_shift, lin0_w, lin0_b, lin1_w, lin1_b, lin2_w, lin2_b):
    raise NotImplementedError("write your pallas kernel here")


# speedup vs baseline: 1.1538x; 1.1538x over previous
import functools

import jax
import jax.numpy as jnp
from jax.experimental import pallas as pl
from jax.experimental.pallas import tpu as pltpu

_CFG = [
    (1, 4, 8, 1, 0.2),
    (4, 8, 8, 1, 0.2),
    (8, 16, 8, 1, 0.2),
    (16, 32, 8, 0, 0.2),
    (32, 64, 7, 1, 0.2),
    (64, 128, 8, 1, 0.2),
    (128, 10, 8, 0, None),
]


def _olen(L, k, p):
    return (L + 2 * p - k) // 4 + 1


def _cmul(w, x):
    if w.shape[1] < 8:
        out = w[:, 0:1] * x[0:1]
        for c in range(1, w.shape[1]):
            out = out + w[:, c : c + 1] * x[c : c + 1]
        return out
    return jnp.dot(w, x, preferred_element_type=jnp.float32)


def _triple_kernel(x_ref, wA_ref, sA_ref, wB_ref, sB_ref, wC_ref, sC_ref,
                   o_ref, *, cinA, kA, alA, LA, kB, pB, alB, LB,
                   kC, pC, alC, LC, GB16, GC):
    h_iota = jax.lax.broadcasted_iota(jnp.int32, (1, GB16), 1)
    g_iota = jax.lax.broadcasted_iota(jnp.int32, (1, GC), 1)

    w16 = []
    for s in range(16):
        acc = None
        for j in range(kA):
            q, t = divmod(4 * s + j, 64)
            if cinA == 1:
                xs = x_ref[t : t + 1, q : q + GB16]
            else:
                xs = x_ref[t, :, q : q + GB16]
            term = _cmul(wA_ref[j], xs)
            acc = term if acc is None else acc + term
        acc = acc + sA_ref[...]
        acc = jnp.where(acc >= 0.0, acc, alA * acc)
        pos = 16 * h_iota + (s - 16)
        w16.append(jnp.where((pos >= 0) & (pos < LA), acc, 0.0))

    z = []
    for r in range(4):
        acc = None
        for j in range(kB):
            q, s = divmod(4 * r + j - 4 * pC - pB + 16, 16)
            term = _cmul(wB_ref[j], w16[s][:, q : q + GC])
            acc = term if acc is None else acc + term
        acc = acc + sB_ref[...]
        acc = jnp.where(acc >= 0.0, acc, alB * acc)
        pos = 4 * g_iota + (r - pC)
        z.append(jnp.where((pos >= 0) & (pos < LB), acc, 0.0))

    acc = None
    for j in range(kC):
        q, r = divmod(j, 4)
        term = _cmul(wC_ref[j], z[r][:, q : q + LC])
        acc = term if acc is None else acc + term
    acc = acc + sC_ref[...]
    o_ref[...] = jnp.where(acc >= 0.0, acc, alC * acc)


def _fold(w, scale):
    return jnp.transpose(w, (2, 0, 1)) * scale[None, :, None]


def _triple(x, pA, pB_, pC_, cfgA, cfgB, cfgC):
    B = x.shape[0]
    cinA, coutA, kA, padA, alA = cfgA
    _, coutB, kB, padB, alB = cfgB
    _, coutC, kC, padC, alC = cfgC
    Lin = x.shape[-1]
    LA = _olen(Lin, kA, padA)
    LB = _olen(LA, kB, padB)
    LC = _olen(LB, kC, padC)
    GC = LC + 2
    GB16 = GC + 2
    GA = GB16 + 2

    left = 64 + padA
    right = 64 * GA - left - Lin
    assert right >= 0
    if cinA == 1:
        xp = jnp.pad(x, ((0, 0), (left, right)))
        xs = xp.reshape(B, GA, 64).transpose(0, 2, 1)
        x_spec = pl.BlockSpec((None, 64, GA), lambda b: (b, 0, 0))
    else:
        xp = jnp.pad(x, ((0, 0), (0, 0), (left, right)))
        xs = xp.reshape(B, cinA, GA, 64).transpose(0, 3, 1, 2)
        x_spec = pl.BlockSpec((None, 64, cinA, GA), lambda b: (b, 0, 0, 0))

    wAk = _fold(pA[0], pA[1])
    wBk = _fold(pB_[0], pB_[1])
    wCk = _fold(pC_[0], pC_[1])
    shA = pA[2].reshape(coutA, 1)
    shB = pB_[2].reshape(coutB, 1)
    shC = pC_[2].reshape(coutC, 1)

    body = functools.partial(
        _triple_kernel, cinA=cinA, kA=kA, alA=alA, LA=LA,
        kB=kB, pB=padB, alB=alB, LB=LB,
        kC=kC, pC=padC, alC=alC, LC=LC, GB16=GB16, GC=GC)

    full = lambda a: pl.BlockSpec(a.shape, lambda b: (0,) * a.ndim)
    return pl.pallas_call(
        body,
        out_shape=jax.ShapeDtypeStruct((B, coutC, LC), jnp.float32),
        grid=(B,),
        in_specs=[x_spec, full(wAk), full(shA), full(wBk), full(shB),
                  full(wCk), full(shC)],
        out_specs=pl.BlockSpec((None, coutC, LC), lambda b: (b, 0, 0)),
        compiler_params=pltpu.CompilerParams(
            dimension_semantics=("parallel",),
            vmem_limit_bytes=64 * 1024 * 1024),
    )(xs, wAk, shA, wBk, shB, wCk, shC)


def _head_kernel(x_ref, w7_ref, w1_ref, b1_ref, w2_ref, b2_ref,
                 w3_ref, b3_ref, o_ref, *, k7, cout7, L7):
    acc = None
    for j in range(k7):
        q, r = divmod(j, 4)
        term = jnp.dot(w7_ref[j], x_ref[r, :, q : q + L7],
                       preferred_element_type=jnp.float32)
        acc = term if acc is None else acc + term
    h = b1_ref[...]
    for c in range(cout7):
        h = h + jnp.dot(acc[c : c + 1, :], w1_ref[c],
                        preferred_element_type=jnp.float32)
    h = jnp.dot(h, w2_ref[...], preferred_element_type=jnp.float32) + b2_ref[...]
    h = jnp.dot(h, w3_ref[...], preferred_element_type=jnp.float32) + b3_ref[...]
    o_ref[...] = jnp.broadcast_to(jax.nn.sigmoid(h[0, 0]), o_ref.shape)


def _head(h6, conv6_w, lin0_w, lin0_b, lin1_w, lin1_b, lin2_w, lin2_b):
    B, cin7, L6 = h6.shape
    _, cout7, k7, p7, _ = _CFG[6]
    L7 = _olen(L6, k7, p7)
    x4 = h6.reshape(B, cin7, L6 // 4, 4).transpose(0, 3, 1, 2)
    w7k = jnp.transpose(conv6_w, (2, 0, 1))
    w1r = lin0_w.reshape(cout7, L7, lin0_w.shape[1])

    body = functools.partial(_head_kernel, k7=k7, cout7=cout7, L7=L7)
    full = lambda a: pl.BlockSpec(a.shape, lambda b: (0,) * a.ndim)
    out = pl.pallas_call(
        body,
        out_shape=jax.ShapeDtypeStruct((B, 8, 128), jnp.float32),
        grid=(B,),
        in_specs=[pl.BlockSpec((None, 4, cin7, L6 // 4), lambda b: (b, 0, 0, 0)),
                  full(w7k), full(w1r), full(lin0_b), full(lin1_w),
                  full(lin1_b), full(lin2_w), full(lin2_b)],
        out_specs=pl.BlockSpec((None, 8, 128), lambda b: (b, 0, 0)),
        compiler_params=pltpu.CompilerParams(
            dimension_semantics=("parallel",),
            vmem_limit_bytes=32 * 1024 * 1024),
    )(x4, w7k, w1r, lin0_b, lin1_w, lin1_b, lin2_w, lin2_b)
    return out[:, 0, :1]


def kernel(x, conv0_w, conv0_scale, conv0_shift, conv1_w, conv1_scale,
           conv1_shift, conv2_w, conv2_scale, conv2_shift, conv3_w,
           conv3_scale, conv3_shift, conv4_w, conv4_scale, conv4_shift,
           conv5_w, conv5_scale, conv5_shift, conv6_w, conv6_scale,
           conv6_shift, lin0_w, lin0_b, lin1_w, lin1_b, lin2_w, lin2_b):
    h2 = _triple(x,
                 (conv0_w, conv0_scale, conv0_shift),
                 (conv1_w, conv1_scale, conv1_shift),
                 (conv2_w, conv2_scale, conv2_shift),
                 _CFG[0], _CFG[1], _CFG[2])
    h6 = _triple(h2,
                 (conv3_w, conv3_scale, conv3_shift),
                 (conv4_w, conv4_scale, conv4_shift),
                 (conv5_w, conv5_scale, conv5_shift),
                 _CFG[3], _CFG[4], _CFG[5])
    out = _head(h6, conv6_w, lin0_w, lin0_b, lin1_w, lin1_b, lin2_w, lin2_b)
    return out, h6
